# baseline (speedup 1.0000x reference)
import functools

import jax
import jax.numpy as jnp
from jax import lax
from jax.experimental import pallas as pl

N = 10000
E = 160000
D = 128
R = 128
LD = 32
LN = 256
ALPHA = 0.2

_NB = 2000


def _tca_body(lf_ref, let_ref, net_ref, wg_ref, ast_ref, mf_ref, h_ref, st_ref):
    lf = lf_ref[...]
    cols = lax.broadcasted_iota(jnp.int32, (1, LN), 1)
    oh = (lf == cols).astype(jnp.float32)
    lemb = jnp.dot(oh, let_ref[...], preferred_element_type=jnp.float32)
    mf = jnp.concatenate([lemb, net_ref[...]], axis=1)
    mf_ref[...] = mf
    h = jnp.dot(mf, wg_ref[...], preferred_element_type=jnp.float32)
    h_ref[...] = h
    st_ref[...] = jnp.dot(h, ast_ref[...], preferred_element_type=jnp.float32)


def _tca(lf2, let, net, wg, ast):
    grid = (N // _NB,)
    return pl.pallas_call(
        _tca_body,
        grid=grid,
        in_specs=[
            pl.BlockSpec((_NB, 1), lambda i: (i, 0)),
            pl.BlockSpec((LN, LD), lambda i: (0, 0)),
            pl.BlockSpec((_NB, D - LD), lambda i: (i, 0)),
            pl.BlockSpec((D, R), lambda i: (0, 0)),
            pl.BlockSpec((R, 2), lambda i: (0, 0)),
        ],
        out_specs=[
            pl.BlockSpec((_NB, D), lambda i: (i, 0)),
            pl.BlockSpec((_NB, R), lambda i: (i, 0)),
            pl.BlockSpec((_NB, 2), lambda i: (i, 0)),
        ],
        out_shape=[
            jax.ShapeDtypeStruct((N, D), jnp.float32),
            jax.ShapeDtypeStruct((N, R), jnp.float32),
            jax.ShapeDtypeStruct((N, 2), jnp.float32),
        ],
    )(lf2, let, net, wg, ast)


def _tcb1_body(hp_ref, rs_ref, ma_ref):
    hp = hp_ref[0] + hp_ref[1]
    rs = jnp.sum(rs_ref[...], axis=0, keepdims=True)
    x = hp / (rs.T + 1e-16)
    x = jnp.where(x > 0, x, jnp.exp(x) - 1.0)
    m = jnp.max(x, axis=0, keepdims=True)
    e = jnp.exp(x - m)
    ma_ref[...] = e / jnp.sum(e, axis=0, keepdims=True)


def _tcb1(hp_part, rs_part):
    return pl.pallas_call(
        _tcb1_body,
        out_shape=jax.ShapeDtypeStruct((N, R), jnp.float32),
    )(hp_part, rs_part)


def _tcb2_body(ma_ref, mf_ref, se_ref):
    k = pl.program_id(0)

    @pl.when(k == 0)
    def _():
        se_ref[...] = jnp.zeros_like(se_ref)

    se_ref[...] += lax.dot_general(
        ma_ref[...], mf_ref[...], (((0,), (0,)), ((), ())),
        preferred_element_type=jnp.float32)


def _tcb2(ma, mf):
    grid = (N // _NB,)
    return pl.pallas_call(
        _tcb2_body,
        grid=grid,
        in_specs=[
            pl.BlockSpec((_NB, R), lambda i: (i, 0)),
            pl.BlockSpec((_NB, D), lambda i: (i, 0)),
        ],
        out_specs=pl.BlockSpec((R, D), lambda i: (0, 0)),
        out_shape=jax.ShapeDtypeStruct((R, D), jnp.float32),
    )(ma, mf)


def _tcc1_body(sip_ref, ma_ref, se_ref, sa_ref, pred_ref):
    k = pl.program_id(0)
    nsteps = pl.num_programs(0)

    @pl.when(k == 0)
    def _():
        sa_ref[...] = jnp.zeros_like(sa_ref)
        se = se_ref[...]
        pred_ref[...] = lax.dot_general(
            se, se, (((1,), (1,)), ((), ())),
            preferred_element_type=jnp.float32)

    si = sip_ref[0] + sip_ref[1]
    sa_ref[...] += lax.dot_general(
        ma_ref[...], si, (((0,), (0,)), ((), ())),
        preferred_element_type=jnp.float32)

    @pl.when(k == nsteps - 1)
    def _():
        sa_ref[...] = jnp.maximum(sa_ref[...] - 0.0001, 0.0)


def _tcc1(si_part, ma, se):
    grid = (N // _NB,)
    return pl.pallas_call(
        _tcc1_body,
        grid=grid,
        in_specs=[
            pl.BlockSpec((2, _NB, R), lambda i: (0, i, 0)),
            pl.BlockSpec((_NB, R), lambda i: (i, 0)),
            pl.BlockSpec((R, D), lambda i: (0, 0)),
        ],
        out_specs=[
            pl.BlockSpec((R, R), lambda i: (0, 0)),
            pl.BlockSpec((R, R), lambda i: (0, 0)),
        ],
        out_shape=[
            jax.ShapeDtypeStruct((R, R), jnp.float32),
            jax.ShapeDtypeStruct((R, R), jnp.float32),
        ],
    )(si_part, ma, se)


def _tcc2_body(sa_ref, se_ref, gw_ref, gb_ref, ret_ref):
    se = se_ref[...]
    support = jnp.dot(se, gw_ref[...], preferred_element_type=jnp.float32)
    dec = jnp.dot(sa_ref[...], support,
                  preferred_element_type=jnp.float32) + gb_ref[...]
    m = jnp.max(dec, axis=0, keepdims=True)
    e = jnp.exp(dec - m)
    da = e / jnp.sum(e, axis=0, keepdims=True)
    ret_ref[...] = lax.dot_general(
        se, da, (((0,), (0,)), ((), ())), preferred_element_type=jnp.float32)


def _tcc2(sa, se, gw, gb2):
    return pl.pallas_call(
        _tcc2_body,
        out_shape=jax.ShapeDtypeStruct((D, N), jnp.float32),
    )(sa, se, gw, gb2)


_EB = 32000


def _tcd_body(pp_ref, out_ref):
    s = jnp.sum(pp_ref[...], axis=0, keepdims=True)
    out_ref[...] = 1.0 / (1.0 + jnp.exp(-s))


def _tcd(pp):
    grid = (2 * E // _EB,)
    return pl.pallas_call(
        _tcd_body,
        grid=grid,
        in_specs=[pl.BlockSpec((32, _EB), lambda i: (0, i))],
        out_specs=pl.BlockSpec((1, _EB), lambda i: (0, i)),
        out_shape=jax.ShapeDtypeStruct((1, 2 * E), jnp.float32),
    )(pp)


def _sparse1(e0, e1, s, t, h):
    edge_e = jnp.exp(-jnp.where((x := s[e0] + t[e1]) > 0, x, ALPHA * x))
    rs = jnp.zeros((N,), jnp.float32).at[e0].add(edge_e)
    hp = jnp.zeros((N, R), jnp.float32).at[e0].add(edge_e[:, None] * h[e1])
    return hp, rs


def _sparse2(e0, e1, ma):
    return jnp.zeros((N, R), jnp.float32).at[e0].add(ma[e1])


def _sparse3(i0, i1, ret):
    g0 = ret[:, i0]
    g1 = ret[:, i1]
    eh = jnp.concatenate([g0, g1], axis=0)
    pp = jnp.sum(g0 * g1, axis=0)
    return eh, jnp.broadcast_to(pp / 32.0, (32, 2 * E))


def kernel(adj_indices, length_feature, node_feature, f_edge, node_emb_table,
           length_emb_table, W_gat, a_gat, gcn_weight, gcn_bias):
    e0 = adj_indices[0].astype(jnp.int32)
    e1 = adj_indices[1].astype(jnp.int32)
    lf2 = length_feature.astype(jnp.int32).reshape(N, 1)
    ast = jnp.concatenate([a_gat[0, :R], a_gat[0, R:]]).reshape(2, R).T

    mf, h, st = _tca(lf2, length_emb_table, node_emb_table, W_gat, ast)
    s, t = st[:, 0], st[:, 1]

    hp, rs = _sparse1(e0, e1, s, t, h)
    hp_part = jnp.stack([hp, jnp.zeros_like(hp)])
    rs_part = jnp.concatenate([rs[None], jnp.zeros((31, N), jnp.float32)])

    ma = _tcb1(hp_part, rs_part)
    se = _tcb2(ma, mf)

    si = _sparse2(e0, e1, ma)
    si_part = jnp.stack([si, jnp.zeros_like(si)])

    sa, pred = _tcc1(si_part, ma, se)
    ret = _tcc2(sa, se, gcn_weight, gcn_bias.reshape(1, N))

    i0 = jnp.concatenate([e0, f_edge[0].astype(jnp.int32)])
    i1 = jnp.concatenate([e1, f_edge[1].astype(jnp.int32)])
    eh, pp = _sparse3(i0, i1, ret)
    ee_out = _tcd(pp).reshape(2 * E)

    edge_label = jnp.concatenate(
        [jnp.ones((E,), jnp.float32), jnp.zeros((E,), jnp.float32)])
    return (eh, sa, pred, ma, ee_out, edge_label)


# speedup vs baseline: 4.5993x; 4.5993x over previous
import functools

import jax
import jax.numpy as jnp
from jax import lax
from jax.experimental import pallas as pl
from jax.experimental.pallas import tpu as pltpu
from jax.experimental.pallas import tpu_sc as plsc

N = 10000
E = 160000
D = 128
R = 128
LD = 32
LN = 256
ALPHA = 0.2

_NB = 2000

_NC, _NS = 2, 16
_NW = _NC * _NS


def _mesh(nc=_NC):
    return plsc.VectorSubcoreMesh(
        core_axis_name="c", subcore_axis_name="s",
        num_cores=nc, num_subcores=_NS)


def _tca_body(lf_ref, let_ref, net_ref, wg_ref, ast_ref, mf_ref, ha_ref,
              st_ref):
    lf = lf_ref[...]
    cols = lax.broadcasted_iota(jnp.int32, (1, LN), 1)
    oh = (lf == cols).astype(jnp.float32)
    lemb = jnp.dot(oh, let_ref[...], preferred_element_type=jnp.float32)
    mf = jnp.concatenate([lemb, net_ref[...]], axis=1)
    mf_ref[...] = mf
    h = jnp.dot(mf, wg_ref[...], preferred_element_type=jnp.float32)
    ha_ref[...] = h
    st_ref[...] = jnp.dot(h, ast_ref[...], preferred_element_type=jnp.float32)


def _tca(lf2, let, net, wg, ast):
    grid = (N // _NB,)
    return pl.pallas_call(
        _tca_body,
        grid=grid,
        in_specs=[
            pl.BlockSpec((_NB, 1), lambda i: (i, 0)),
            pl.BlockSpec((LN, LD), lambda i: (0, 0)),
            pl.BlockSpec((_NB, D - LD), lambda i: (i, 0)),
            pl.BlockSpec((D, R), lambda i: (0, 0)),
            pl.BlockSpec((R, 2), lambda i: (0, 0)),
        ],
        out_specs=[
            pl.BlockSpec((_NB, D), lambda i: (i, 0)),
            pl.BlockSpec((_NB, R), lambda i: (i, 0)),
            pl.BlockSpec((_NB, 2), lambda i: (i, 0)),
        ],
        out_shape=[
            jax.ShapeDtypeStruct((N, D), jnp.float32),
            jax.ShapeDtypeStruct((N, R), jnp.float32),
            jax.ShapeDtypeStruct((N, 2), jnp.float32),
        ],
    )(lf2, let, net, wg, ast)


_EBATCH = 128
_NBATCH1 = E // _EBATCH
_KMAX1 = -(-_NBATCH1 // _NW)
_KMAX1S = -(-_NBATCH1 // _NS)
_NPAD = 10240
_ZR = _NPAD // _NS
_OR = 624


_HLF = 5120
_HROWS = _HLF + 128
_ZT = _HROWS // _NS
_OT = _HLF // _NS
_RSROWS = 48


def _sc1(e0r, e1r, s1, t1, h, base):
    @functools.partial(
        pl.kernel,
        out_type=[
            jax.ShapeDtypeStruct((_NC, _HLF, R), jnp.float32),
            jax.ShapeDtypeStruct((_NC, _RSROWS, 128), jnp.float32),
        ],
        mesh=_mesh(),
        compiler_params=pltpu.CompilerParams(needs_layout_passes=False),
        scratch_types=[
            pltpu.VMEM((128, 128), jnp.float32),
            pltpu.VMEM((128, 128), jnp.float32),
            pltpu.VMEM((_EBATCH,), jnp.int32),
            pltpu.VMEM((_EBATCH,), jnp.int32),
            pltpu.VMEM((_EBATCH,), jnp.int32),
            pltpu.VMEM((_EBATCH,), jnp.int32),
            pltpu.VMEM((_EBATCH,), jnp.float32),
            pltpu.VMEM((_EBATCH, R), jnp.float32),
            pltpu.VMEM((_EBATCH, 128), jnp.float32),
            pltpu.VMEM((128, R), jnp.float32),
            pltpu.VMEM_SHARED((_HROWS, R), jnp.float32),
            pltpu.VMEM_SHARED((_RSROWS, 128), jnp.float32),
            pltpu.SemaphoreType.DMA,
        ],
    )
    def k(e0r_h, e1r_h, s_h, t_h, h_h, out_h, rsout_h,
          s_v, t_v, e0b, e1b, hpidx, rsidx, eeb, rows, rsrows, zbuf,
          hp_sh, rs_sh, sem):
        cid = lax.axis_index("c")
        sid = lax.axis_index("s")
        w = cid * _NS + sid
        pltpu.sync_copy(s_h, s_v)
        pltpu.sync_copy(t_h, t_v)

        def zb(j, _):
            for c in range(R // 16):
                zbuf[j, pl.ds(c * 16, 16)] = jnp.zeros((16,), jnp.float32)
            return 0

        lax.fori_loop(0, 128, zb, 0)
        for i in range(2):
            pltpu.sync_copy(zbuf, hp_sh.at[pl.ds(sid * _ZT + i * 128, 128)])
        pltpu.sync_copy(zbuf.at[pl.ds(0, _ZT - 256)],
                        hp_sh.at[pl.ds(sid * _ZT + 256, _ZT - 256)])

        def zr(j, _):
            for c in range(128 // 16):
                rsrows[j, pl.ds(c * 16, 16)] = jnp.zeros((16,), jnp.float32)
            return 0

        lax.fori_loop(0, _EBATCH, zr, 0)

        @pl.when(sid == 0)
        def _():
            pltpu.sync_copy(zbuf.at[pl.ds(0, _RSROWS)], rs_sh)

        plsc.subcore_barrier()

        def batch(kk, _):
            g = kk * _NW + w

            @pl.when(g < _NBATCH1)
            def _():
                pltpu.sync_copy(e0r_h.at[g], e0b)
                pltpu.sync_copy(e1r_h.at[g], e1b)
                for c in range(_EBATCH // 16):
                    i0 = e0b[pl.ds(c * 16, 16)]
                    i1 = e1b[pl.ds(c * 16, 16)]
                    x = (plsc.load_gather(s_v, [i0 >> 7, i0 & 127]) +
                         plsc.load_gather(t_v, [i1 >> 7, i1 & 127]))
                    x = jnp.where(x > 0, x, ALPHA * x)
                    ee = jnp.exp(-x)
                    eeb[pl.ds(c * 16, 16)] = ee
                    ih = i0 - base
                    valid = (ih >= 0) & (ih < _HLF)
                    hpidx[pl.ds(c * 16, 16)] = jnp.where(valid, ih, _HLF)
                    rsidx[pl.ds(c * 16, 16)] = jnp.where(
                        valid, ih >> 7, _HLF >> 7)
                    jrow = lax.iota(jnp.int32, 16) + (c * 16)
                    plsc.store_scatter(rsrows, [jrow, ih & 127], ee)
                pltpu.async_copy(h_h.at[e1b], rows, sem).wait()

                def sc(jo, _):
                    ee16 = eeb[pl.ds(jo * 16, 16)]
                    for jl in range(16):
                        ee = jnp.full((16,), ee16[jl], jnp.float32)
                        j = jo * 16 + jl
                        for c in range(R // 16):
                            rows[j, pl.ds(c * 16, 16)] = (
                                rows[j, pl.ds(c * 16, 16)] * ee)
                    return 0

                lax.fori_loop(0, _EBATCH // 16, sc, 0)
                pltpu.sync_copy(rows, hp_sh.at[hpidx], add=True)
                pltpu.sync_copy(rsrows, rs_sh.at[rsidx], add=True)
                for c in range(_EBATCH // 16):
                    i0 = e0b[pl.ds(c * 16, 16)]
                    ih = i0 - base
                    jrow = lax.iota(jnp.int32, 16) + (c * 16)
                    plsc.store_scatter(
                        rsrows, [jrow, ih & 127], jnp.zeros((16,), jnp.float32))
            return 0

        lax.fori_loop(0, _KMAX1, batch, 0)
        plsc.subcore_barrier()
        pltpu.sync_copy(hp_sh.at[pl.ds(sid * _OT, _OT)],
                        out_h.at[cid, pl.ds(sid * _OT, _OT)])

        @pl.when(sid == 0)
        def _():
            pltpu.sync_copy(rs_sh, rsout_h.at[cid])

    return k(e0r, e1r, s1, t1, h)


def _sc2(e0r, e1r, ma):
    @functools.partial(
        pl.kernel,
        out_type=[
            jax.ShapeDtypeStruct((E, R), jnp.float32),
            jax.ShapeDtypeStruct((E, R), jnp.float32),
        ],
        mesh=_mesh(),
        compiler_params=pltpu.CompilerParams(needs_layout_passes=False),
        scratch_types=[
            pltpu.VMEM((_EBATCH,), jnp.int32),
            pltpu.VMEM((_EBATCH,), jnp.int32),
            pltpu.VMEM((_EBATCH, R), jnp.float32),
            pltpu.VMEM((_EBATCH, R), jnp.float32),
            pltpu.SemaphoreType.DMA,
            pltpu.SemaphoreType.DMA,
        ],
    )
    def k(e0r_h, e1r_h, ma_h, m0_h, m1_h, e0b, e1b, rows0, rows1, sem,
          sem_out):
        cid = lax.axis_index("c")
        sid = lax.axis_index("s")
        w = cid * _NS + sid

        def batch(kk, _):
            g = kk * _NW + w

            @pl.when(g < _NBATCH1)
            def _():
                pltpu.sync_copy(e0r_h.at[g], e0b)
                pltpu.sync_copy(e1r_h.at[g], e1b)
                cp0 = pltpu.async_copy(ma_h.at[e0b], rows0, sem)
                cp1 = pltpu.async_copy(ma_h.at[e1b], rows1, sem)
                cp0.wait()
                cp1.wait()
                pltpu.async_copy(
                    rows0, m0_h.at[pl.ds(g * _EBATCH, _EBATCH)], sem_out)
                pltpu.async_copy(
                    rows1, m1_h.at[pl.ds(g * _EBATCH, _EBATCH)], sem_out)
                pltpu.make_async_copy(
                    rows0, m0_h.at[pl.ds(g * _EBATCH, _EBATCH)],
                    sem_out).wait()
                pltpu.make_async_copy(
                    rows1, m1_h.at[pl.ds(g * _EBATCH, _EBATCH)],
                    sem_out).wait()
            return 0

        lax.fori_loop(0, _KMAX1, batch, 0)

    return k(e0r, e1r, ma)


_EB3 = 3200
_NB3 = (2 * E) // _EB3
_CH3 = _EB3 // 16
_RPT = R // _NW


def _sc3(ret, i0, i1):
    @functools.partial(
        pl.kernel,
        out_type=[
            jax.ShapeDtypeStruct((2 * R, 2 * E), jnp.float32),
            jax.ShapeDtypeStruct((_NW, 2 * E), jnp.float32),
        ],
        mesh=_mesh(),
        compiler_params=pltpu.CompilerParams(needs_layout_passes=False),
        scratch_types=[
            pltpu.VMEM((_RPT, 80, 128), jnp.float32),
        ] + [pltpu.VMEM((_EB3,), jnp.int32) for _ in range(4)]
          + [pltpu.VMEM((_EB3,), jnp.float32) for _ in range(18)] + [
            pltpu.SemaphoreType.DMA,
            pltpu.SemaphoreType.DMA,
        ],
    )
    def k(ret_h, i0_h, i1_h, eh_h, pp_h, tab,
          i0va, i0vb, i1va, i1vb,
          g000, g010, g100, g110, g200, g210, g300, g310, acc0,
          g001, g011, g101, g111, g201, g211, g301, g311, acc1,
          sem_in, sem_out):
        cid = lax.axis_index("c")
        sid = lax.axis_index("s")
        w = cid * _NS + sid
        r0 = w * _RPT
        i0v = (i0va, i0vb)
        i1v = (i1va, i1vb)
        gb = (((g000, g010), (g100, g110), (g200, g210), (g300, g310)),
              ((g001, g011), (g101, g111), (g201, g211), (g301, g311)))
        accb = (acc0, acc1)
        pltpu.sync_copy(ret_h.at[w], tab)
        pltpu.async_copy(i0_h.at[pl.ds(0, _EB3)], i0v[0], sem_in)
        pltpu.async_copy(i1_h.at[pl.ds(0, _EB3)], i1v[0], sem_in)

        def pair(kk, _):
            for p in range(2):
                b = kk * 2 + p
                pltpu.make_async_copy(
                    i0_h.at[pl.ds(0, _EB3)], i0v[p], sem_in).wait()
                pltpu.make_async_copy(
                    i1_h.at[pl.ds(0, _EB3)], i1v[p], sem_in).wait()

                @pl.when(b + 1 < _NB3)
                def _():
                    off = (b + 1) * _EB3
                    pltpu.async_copy(
                        i0_h.at[pl.ds(off, _EB3)], i0v[1 - p], sem_in)
                    pltpu.async_copy(
                        i1_h.at[pl.ds(off, _EB3)], i1v[1 - p], sem_in)

                @pl.when(b >= 2)
                def _():
                    for _i in range(9):
                        pltpu.make_async_copy(
                            i0_h.at[pl.ds(0, _EB3)], accb[p], sem_out).wait()

                def cj(j, _):
                    idx0 = i0v[p][pl.ds(j * 16, 16)]
                    idx1 = i1v[p][pl.ds(j * 16, 16)]
                    i0hi, i0lo = idx0 >> 7, idx0 & 127
                    i1hi, i1lo = idx1 >> 7, idx1 & 127
                    acc = jnp.zeros((16,), jnp.float32)
                    for rl in range(_RPT):
                        rli = jnp.full((16,), rl, jnp.int32)
                        g0 = plsc.load_gather(tab, [rli, i0hi, i0lo])
                        g1 = plsc.load_gather(tab, [rli, i1hi, i1lo])
                        gb[p][rl][0][pl.ds(j * 16, 16)] = g0
                        gb[p][rl][1][pl.ds(j * 16, 16)] = g1
                        acc = acc + g0 * g1
                    accb[p][pl.ds(j * 16, 16)] = acc
                    return 0

                lax.fori_loop(0, _CH3, cj, 0)
                off = b * _EB3
                for rl in range(_RPT):
                    pltpu.async_copy(
                        gb[p][rl][0], eh_h.at[r0 + rl, pl.ds(off, _EB3)],
                        sem_out)
                    pltpu.async_copy(
                        gb[p][rl][1],
                        eh_h.at[R + r0 + rl, pl.ds(off, _EB3)], sem_out)
                pltpu.async_copy(accb[p], pp_h.at[w, pl.ds(off, _EB3)],
                                 sem_out)
            return 0

        lax.fori_loop(0, _NB3 // 2, pair, 0)
        for _i in range(18):
            pltpu.make_async_copy(
                i0_h.at[pl.ds(0, _EB3)], acc0, sem_out).wait()

    return k(ret, i0, i1)


def _tcb1_body(hpa_ref, hpb_ref, rs_ref, ma_ref):
    hp = jnp.concatenate(
        [hpa_ref[0] + hpa_ref[1], (hpb_ref[0] + hpb_ref[1])[:N - _HLF]],
        axis=0)
    x = hp / (rs_ref[...] + 1e-16)
    x = jnp.where(x > 0, x, jnp.exp(x) - 1.0)
    m = jnp.max(x, axis=0, keepdims=True)
    e = jnp.exp(x - m)
    ma_ref[...] = e / jnp.sum(e, axis=0, keepdims=True)


def _tcb1(hpa, hpb, rs2):
    return pl.pallas_call(
        _tcb1_body,
        out_shape=jax.ShapeDtypeStruct((N, R), jnp.float32),
    )(hpa, hpb, rs2)


def _tcb2_body(ma_ref, mf_ref, se_ref):
    k = pl.program_id(0)

    @pl.when(k == 0)
    def _():
        se_ref[...] = jnp.zeros_like(se_ref)

    se_ref[...] += lax.dot_general(
        ma_ref[...], mf_ref[...], (((0,), (0,)), ((), ())),
        preferred_element_type=jnp.float32)


def _tcb2(ma, mf):
    grid = (N // _NB,)
    return pl.pallas_call(
        _tcb2_body,
        grid=grid,
        in_specs=[
            pl.BlockSpec((_NB, R), lambda i: (i, 0)),
            pl.BlockSpec((_NB, D), lambda i: (i, 0)),
        ],
        out_specs=pl.BlockSpec((R, D), lambda i: (0, 0)),
        out_shape=jax.ShapeDtypeStruct((R, D), jnp.float32),
    )(ma, mf)


_EBK = 8000


def _tcc1_body(m0_ref, m1_ref, se_ref, sa_ref, pred_ref):
    k = pl.program_id(0)
    nsteps = pl.num_programs(0)

    @pl.when(k == 0)
    def _():
        sa_ref[...] = jnp.zeros_like(sa_ref)
        se = se_ref[...]
        pred_ref[...] = lax.dot_general(
            se, se, (((1,), (1,)), ((), ())),
            preferred_element_type=jnp.float32)

    sa_ref[...] += lax.dot_general(
        m0_ref[...], m1_ref[...], (((0,), (0,)), ((), ())),
        preferred_element_type=jnp.float32)

    @pl.when(k == nsteps - 1)
    def _():
        sa_ref[...] = jnp.maximum(sa_ref[...] - 0.0001, 0.0)


def _tcc1(m0, m1, se):
    grid = (E // _EBK,)
    return pl.pallas_call(
        _tcc1_body,
        grid=grid,
        in_specs=[
            pl.BlockSpec((_EBK, R), lambda i: (i, 0)),
            pl.BlockSpec((_EBK, R), lambda i: (i, 0)),
            pl.BlockSpec((R, D), lambda i: (0, 0)),
        ],
        out_specs=[
            pl.BlockSpec((R, R), lambda i: (0, 0)),
            pl.BlockSpec((R, R), lambda i: (0, 0)),
        ],
        out_shape=[
            jax.ShapeDtypeStruct((R, R), jnp.float32),
            jax.ShapeDtypeStruct((R, R), jnp.float32),
        ],
    )(m0, m1, se)


def _tcc2_body(sa_ref, se_ref, gw_ref, gb_ref, ret_ref):
    se = se_ref[...]
    support = jnp.dot(se, gw_ref[...], preferred_element_type=jnp.float32)
    dec = jnp.dot(sa_ref[...], support,
                  preferred_element_type=jnp.float32) + gb_ref[...]
    m = jnp.max(dec, axis=0, keepdims=True)
    e = jnp.exp(dec - m)
    da = e / jnp.sum(e, axis=0, keepdims=True)
    ret_ref[...] = lax.dot_general(
        se, da, (((0,), (0,)), ((), ())), preferred_element_type=jnp.float32)


def _tcc2(sa, se, gw, gb2):
    return pl.pallas_call(
        _tcc2_body,
        out_shape=jax.ShapeDtypeStruct((D, N), jnp.float32),
    )(sa, se, gw, gb2)


_EB = 32000


def _tcd_body(pp_ref, out_ref):
    s = jnp.sum(pp_ref[...], axis=0, keepdims=True)
    out_ref[...] = 1.0 / (1.0 + jnp.exp(-s))


def _tcd(pp):
    grid = (2 * E // _EB,)
    return pl.pallas_call(
        _tcd_body,
        grid=grid,
        in_specs=[pl.BlockSpec((_NW, _EB), lambda i: (0, i))],
        out_specs=pl.BlockSpec((1, _EB), lambda i: (0, i)),
        out_shape=jax.ShapeDtypeStruct((1, 2 * E), jnp.float32),
    )(pp)


def kernel(adj_indices, length_feature, node_feature, f_edge, node_emb_table,
           length_emb_table, W_gat, a_gat, gcn_weight, gcn_bias):
    e0 = adj_indices[0].astype(jnp.int32)
    e1 = adj_indices[1].astype(jnp.int32)
    e0r = e0.reshape(_NBATCH1, _EBATCH)
    e1r = e1.reshape(_NBATCH1, _EBATCH)
    lf2 = length_feature.astype(jnp.int32).reshape(N, 1)
    ast = jnp.concatenate([a_gat[0, :R], a_gat[0, R:]]).reshape(2, R).T

    mf, h, st = _tca(lf2, length_emb_table, node_emb_table, W_gat, ast)
    stp = jnp.pad(st, ((0, 16384 - N), (0, 0)))
    s1 = stp[:, 0].reshape(128, 128)
    t1 = stp[:, 1].reshape(128, 128)

    hpa, rsa = _sc1(e0r, e1r, s1, t1, h, 0)
    hpb, rsb = _sc1(e0r, e1r, s1, t1, h, _HLF)
    rs_full = jnp.concatenate([
        (rsa[0] + rsa[1]).reshape(_RSROWS * 128)[:_HLF],
        (rsb[0] + rsb[1]).reshape(_RSROWS * 128)[:N - _HLF],
    ]).reshape(N, 1)
    ma = _tcb1(hpa, hpb, rs_full)
    se = _tcb2(ma, mf)

    m0, m1 = _sc2(e0r, e1r, ma)
    sa, pred = _tcc1(m0, m1, se)
    ret = _tcc2(sa, se, gcn_weight, gcn_bias.reshape(1, N))
    retp = jnp.pad(ret, ((0, 0), (0, 10240 - N))).reshape(_NW, _RPT, 80, 128)

    i0 = jnp.concatenate([e0, f_edge[0].astype(jnp.int32)])
    i1 = jnp.concatenate([e1, f_edge[1].astype(jnp.int32)])
    eh, pp = _sc3(retp, i0, i1)
    ee_out = _tcd(pp).reshape(2 * E)

    edge_label = jnp.concatenate(
        [jnp.ones((E,), jnp.float32), jnp.zeros((E,), jnp.float32)])
    return (eh, sa, pred, ma, ee_out, edge_label)
